# baseline (speedup 1.0000x reference)
import functools

import jax
import jax.numpy as jnp
from jax import lax
from jax.experimental import pallas as pl
from jax.experimental.pallas import tpu as pltpu
from jax.experimental.pallas import tpu_sc as plsc

N = 10000
E = 160000
D = 256
DH = 128
NPAD = 10240
EPAD = 163840
NC, NS = 2, 16
ROWS_PER_TILE = NPAD // NS
EDGES_PER_TILE = EPAD // NS
ECHUNK = 80
NCHUNKS = EDGES_PER_TILE // ECHUNK
NBUF = 2
NPAIRS = NCHUNKS // NBUF
DEG_CHUNK = 128
DEG_EPT = EPAD // (NC * NS)
DEG_NCHUNKS = DEG_EPT // DEG_CHUNK
DEG_NPAIRS = DEG_NCHUNKS // NBUF

_MESH = plsc.VectorSubcoreMesh(core_axis_name="c", subcore_axis_name="s")


@functools.partial(
    pl.kernel,
    out_type=jax.ShapeDtypeStruct((NC * NPAD,), jnp.float32),
    mesh=_MESH,
    scratch_types=[
        pltpu.VMEM((DEG_EPT,), jnp.int32),
        [pltpu.VMEM((DEG_CHUNK,), jnp.int32) for _ in range(NBUF)],
        pltpu.VMEM((DEG_CHUNK,), jnp.float32),
        pltpu.VMEM_SHARED((NPAD,), jnp.float32),
        pltpu.SemaphoreType.DMA((NBUF,)),
    ],
)
def _deg_kernel(dst_hbm, zeros_hbm, ones_hbm, out_hbm, dstb, didx, ones_v,
                deg_sp, ssem):
    c = lax.axis_index("c")
    s = lax.axis_index("s")
    tid = c * NS + s
    r0 = s * ROWS_PER_TILE
    pltpu.sync_copy(zeros_hbm.at[pl.ds(r0, ROWS_PER_TILE)],
                    deg_sp.at[pl.ds(r0, ROWS_PER_TILE)])
    pltpu.sync_copy(ones_hbm, ones_v)
    e_base = pl.multiple_of(tid * DEG_EPT, 8)
    pltpu.sync_copy(dst_hbm.at[pl.ds(e_base, DEG_EPT)], dstb)
    plsc.subcore_barrier()

    def pair(k, carry):
        for b in range(NBUF):
            e0 = pl.multiple_of((k * NBUF + b) * DEG_CHUNK, 16)

            @pl.when(k > 0)
            def _(b=b):
                pltpu.make_async_copy(ones_hbm, ones_v, ssem.at[b]).wait()

            for j in range(DEG_CHUNK // 16):
                o = pl.multiple_of(e0 + j * 16, 16)
                didx[b][pl.ds(j * 16, 16)] = dstb[pl.ds(o, 16)]
            pltpu.async_copy(ones_v, deg_sp.at[didx[b]], ssem.at[b], add=True)
        return carry

    lax.fori_loop(0, DEG_NPAIRS, pair, 0)
    for b in range(NBUF):
        pltpu.make_async_copy(ones_hbm, ones_v, ssem.at[b]).wait()
    plsc.subcore_barrier()
    out0 = c * NPAD + r0
    pltpu.sync_copy(deg_sp.at[pl.ds(r0, ROWS_PER_TILE)],
                    out_hbm.at[pl.ds(out0, ROWS_PER_TILE)])


@functools.partial(
    pl.kernel,
    out_type=[
        jax.ShapeDtypeStruct((NPAD, DH), jnp.float32),
        jax.ShapeDtypeStruct((NPAD, DH), jnp.float32),
    ],
    mesh=_MESH,
    scratch_types=[
        pltpu.VMEM((EDGES_PER_TILE,), jnp.int32),
        pltpu.VMEM((EDGES_PER_TILE,), jnp.int32),
        [pltpu.VMEM((ECHUNK, DH), jnp.float32) for _ in range(NBUF)],
        [pltpu.VMEM((ECHUNK,), jnp.int32) for _ in range(NBUF)],
        [pltpu.VMEM((ECHUNK,), jnp.int32) for _ in range(NBUF)],
        pltpu.VMEM_SHARED((NPAD, DH), jnp.float32),
        pltpu.SemaphoreType.DMA((NBUF,)),
        pltpu.SemaphoreType.DMA((NBUF,)),
    ],
)
def _scatter_kernel(gl_hbm, gr_hbm, src_hbm, dst_hbm, outl_hbm, outr_hbm,
                    srcb, dstb, rows, sidx, didx, acc_sp, gsem, ssem):
    c = lax.axis_index("c")
    s = lax.axis_index("s")
    r0 = s * ROWS_PER_TILE

    def run(g_hbm, out_hbm):
        pltpu.sync_copy(g_hbm.at[pl.ds(r0, ROWS_PER_TILE)],
                        acc_sp.at[pl.ds(r0, ROWS_PER_TILE)])
        plsc.subcore_barrier()

        def load_idx(b, e0):
            for j in range(ECHUNK // 16):
                o = pl.multiple_of(e0 + j * 16, 16)
                sidx[b][pl.ds(j * 16, 16)] = srcb[pl.ds(o, 16)]
                didx[b][pl.ds(j * 16, 16)] = dstb[pl.ds(o, 16)]

        e_base = pl.multiple_of(s * EDGES_PER_TILE, 8)
        pltpu.sync_copy(src_hbm.at[pl.ds(e_base, EDGES_PER_TILE)], srcb)
        pltpu.sync_copy(dst_hbm.at[pl.ds(e_base, EDGES_PER_TILE)], dstb)

        def pair(k, carry):
            descs = []
            for b in range(NBUF):
                e0 = pl.multiple_of((k * NBUF + b) * ECHUNK, 16)

                @pl.when(k > 0)
                def _(b=b):
                    pltpu.make_async_copy(
                        g_hbm.at[pl.ds(0, ECHUNK)], rows[b],
                        ssem.at[b]).wait()

                load_idx(b, e0)
                descs.append(
                    pltpu.async_copy(g_hbm.at[sidx[b]], rows[b], gsem.at[b])
                )
            for b in range(NBUF):
                descs[b].wait()
                pltpu.async_copy(rows[b], acc_sp.at[didx[b]],
                                 ssem.at[b], add=True)
            return carry

        lax.fori_loop(0, NPAIRS, pair, 0)

        for b in range(NBUF):
            pltpu.make_async_copy(
                g_hbm.at[pl.ds(0, ECHUNK)], rows[b], ssem.at[b]
            ).wait()
        plsc.subcore_barrier()
        pltpu.sync_copy(acc_sp.at[pl.ds(r0, ROWS_PER_TILE)],
                        out_hbm.at[pl.ds(r0, ROWS_PER_TILE)])

    @pl.when(c == 0)
    def _():
        run(gl_hbm, outl_hbm)

    @pl.when(c == 1)
    def _():
        run(gr_hbm, outr_hbm)


_TCBLK = 512


def _tca_body(x_ref, w_ref, da_ref, db_ref, gl_ref, gr_ref, dis_ref):
    deg = da_ref[...] + db_ref[...] + 1.0
    dis = lax.rsqrt(deg)
    h = jnp.dot(x_ref[...], w_ref[...], preferred_element_type=jnp.float32)
    g = h * dis
    gl_ref[...] = g[:, :DH]
    gr_ref[...] = g[:, DH:]
    dis_ref[...] = dis


def _tc_transform(x_pad, W, degA, degB):
    grid = (NPAD // _TCBLK,)
    return pl.pallas_call(
        _tca_body,
        grid=grid,
        in_specs=[
            pl.BlockSpec((_TCBLK, D), lambda i: (i, 0)),
            pl.BlockSpec((D, D), lambda i: (0, 0)),
            pl.BlockSpec((_TCBLK, 1), lambda i: (i, 0)),
            pl.BlockSpec((_TCBLK, 1), lambda i: (i, 0)),
        ],
        out_specs=[
            pl.BlockSpec((_TCBLK, DH), lambda i: (i, 0)),
            pl.BlockSpec((_TCBLK, DH), lambda i: (i, 0)),
            pl.BlockSpec((_TCBLK, 1), lambda i: (i, 0)),
        ],
        out_shape=[
            jax.ShapeDtypeStruct((NPAD, DH), jnp.float32),
            jax.ShapeDtypeStruct((NPAD, DH), jnp.float32),
            jax.ShapeDtypeStruct((NPAD, 1), jnp.float32),
        ],
    )(x_pad, W, degA, degB)


def _tcb_body(al_ref, ar_ref, dis_ref, b_ref, o_ref):
    dis = dis_ref[...]
    b = b_ref[...]
    o_ref[:, :DH] = jnp.tanh(al_ref[...] * dis + b[:, :DH])
    o_ref[:, DH:] = jnp.tanh(ar_ref[...] * dis + b[:, DH:])


def _tc_final(accL, accR, dis, b2d):
    grid = (NPAD // _TCBLK,)
    return pl.pallas_call(
        _tcb_body,
        grid=grid,
        in_specs=[
            pl.BlockSpec((_TCBLK, DH), lambda i: (i, 0)),
            pl.BlockSpec((_TCBLK, DH), lambda i: (i, 0)),
            pl.BlockSpec((_TCBLK, 1), lambda i: (i, 0)),
            pl.BlockSpec((1, D), lambda i: (0, 0)),
        ],
        out_specs=pl.BlockSpec((_TCBLK, D), lambda i: (i, 0)),
        out_shape=jax.ShapeDtypeStruct((NPAD, D), jnp.float32),
    )(accL, accR, dis, b2d)


@jax.jit
def kernel(x, edge_index, W, b):
    src = edge_index[0].astype(jnp.int32)
    dst = edge_index[1].astype(jnp.int32)
    pad_idx = jnp.full((EPAD - E,), NPAD - 1, jnp.int32)
    src = jnp.concatenate([src, pad_idx])
    dst = jnp.concatenate([dst, pad_idx])
    x_pad = jnp.pad(x, ((0, NPAD - N), (0, 0)))
    zeros = jnp.zeros((NPAD,), jnp.float32)
    ones = jnp.ones((DEG_CHUNK,), jnp.float32)

    deg2 = _deg_kernel(dst, zeros, ones)
    degA = deg2[:NPAD].reshape(NPAD, 1)
    degB = deg2[NPAD:].reshape(NPAD, 1)

    gl, gr, dis = _tc_transform(x_pad, W, degA, degB)
    accL, accR = _scatter_kernel(gl, gr, src, dst)
    out = _tc_final(accL, accR, dis, b.reshape(1, D))
    return out[:N]


# speedup vs baseline: 1.6577x; 1.6577x over previous
import functools

import jax
import jax.numpy as jnp
from jax import lax
from jax.experimental import pallas as pl
from jax.experimental.pallas import tpu as pltpu
from jax.experimental.pallas import tpu_sc as plsc

N = 10000
E = 160000
D = 256
DH = 128
NPAD = 10240
EPAD = 163840
NC, NS = 2, 16
ROWS_PER_TILE = NPAD // NS
EDGES_PER_TILE = EPAD // NS
ECHUNK = 80
NCHUNKS = EDGES_PER_TILE // ECHUNK
NBUF = 2
NPAIRS = NCHUNKS // NBUF
DEG_CHUNK = 128
DEG_EPT = EPAD // (NC * NS)
DEG_NCHUNKS = DEG_EPT // DEG_CHUNK
DEG_NPAIRS = DEG_NCHUNKS // NBUF

_MESH = plsc.VectorSubcoreMesh(core_axis_name="c", subcore_axis_name="s")


@functools.partial(
    pl.kernel,
    out_type=jax.ShapeDtypeStruct((NC * NPAD,), jnp.float32),
    mesh=_MESH,
    scratch_types=[
        pltpu.VMEM((DEG_EPT,), jnp.int32),
        [pltpu.VMEM((DEG_CHUNK,), jnp.int32) for _ in range(NBUF)],
        pltpu.VMEM((DEG_CHUNK,), jnp.float32),
        pltpu.VMEM_SHARED((NPAD,), jnp.float32),
        pltpu.SemaphoreType.DMA((NBUF,)),
    ],
)
def _deg_kernel(dst_hbm, zeros_hbm, ones_hbm, out_hbm, dstb, didx, ones_v,
                deg_sp, ssem):
    c = lax.axis_index("c")
    s = lax.axis_index("s")
    tid = c * NS + s
    r0 = s * ROWS_PER_TILE
    pltpu.sync_copy(zeros_hbm.at[pl.ds(r0, ROWS_PER_TILE)],
                    deg_sp.at[pl.ds(r0, ROWS_PER_TILE)])
    pltpu.sync_copy(ones_hbm, ones_v)
    e_base = pl.multiple_of(tid * DEG_EPT, 8)
    pltpu.sync_copy(dst_hbm.at[pl.ds(e_base, DEG_EPT)], dstb)
    plsc.subcore_barrier()

    def pair(k, carry):
        for b in range(NBUF):
            e0 = pl.multiple_of((k * NBUF + b) * DEG_CHUNK, 16)

            @pl.when(k > 0)
            def _(b=b):
                pltpu.make_async_copy(ones_hbm, ones_v, ssem.at[b]).wait()

            for j in range(DEG_CHUNK // 16):
                o = pl.multiple_of(e0 + j * 16, 16)
                didx[b][pl.ds(j * 16, 16)] = dstb[pl.ds(o, 16)]
            pltpu.async_copy(ones_v, deg_sp.at[didx[b]], ssem.at[b], add=True)
        return carry

    lax.fori_loop(0, DEG_NPAIRS, pair, 0)
    for b in range(NBUF):
        pltpu.make_async_copy(ones_hbm, ones_v, ssem.at[b]).wait()
    plsc.subcore_barrier()
    out0 = c * NPAD + r0
    pltpu.sync_copy(deg_sp.at[pl.ds(r0, ROWS_PER_TILE)],
                    out_hbm.at[pl.ds(out0, ROWS_PER_TILE)])


@functools.partial(
    pl.kernel,
    out_type=[
        jax.ShapeDtypeStruct((NPAD, DH), jnp.float32),
        jax.ShapeDtypeStruct((NPAD, DH), jnp.float32),
    ],
    mesh=_MESH,
    scratch_types=[
        pltpu.VMEM((EDGES_PER_TILE,), jnp.int32),
        pltpu.VMEM((EDGES_PER_TILE,), jnp.int32),
        [pltpu.VMEM((ECHUNK, DH), jnp.float32) for _ in range(NBUF)],
        [pltpu.VMEM((ECHUNK,), jnp.int32) for _ in range(NBUF)],
        [pltpu.VMEM((ECHUNK,), jnp.int32) for _ in range(NBUF)],
        pltpu.VMEM_SHARED((NPAD, DH), jnp.float32),
        pltpu.SemaphoreType.DMA((NBUF,)),
        pltpu.SemaphoreType.DMA((NBUF,)),
    ],
)
def _scatter_kernel(gl_hbm, gr_hbm, src_hbm, dst_hbm, outl_hbm, outr_hbm,
                    srcb, dstb, rows, sidx, didx, acc_sp, gsem, ssem):
    c = lax.axis_index("c")
    s = lax.axis_index("s")
    r0 = s * ROWS_PER_TILE

    def run(g_hbm, out_hbm):
        pltpu.sync_copy(g_hbm.at[pl.ds(r0, ROWS_PER_TILE)],
                        acc_sp.at[pl.ds(r0, ROWS_PER_TILE)])
        plsc.subcore_barrier()

        def load_idx(b, e0):
            for j in range(ECHUNK // 16):
                o = pl.multiple_of(e0 + j * 16, 16)
                sidx[b][pl.ds(j * 16, 16)] = srcb[pl.ds(o, 16)]
                didx[b][pl.ds(j * 16, 16)] = dstb[pl.ds(o, 16)]

        e_base = pl.multiple_of(s * EDGES_PER_TILE, 8)
        pltpu.sync_copy(src_hbm.at[pl.ds(e_base, EDGES_PER_TILE)], srcb)
        pltpu.sync_copy(dst_hbm.at[pl.ds(e_base, EDGES_PER_TILE)], dstb)

        def pair(k, carry):
            descs = []
            for b in range(NBUF):
                e0 = pl.multiple_of((k * NBUF + b) * ECHUNK, 16)

                @pl.when(k > 0)
                def _(b=b):
                    pltpu.make_async_copy(
                        g_hbm.at[pl.ds(0, ECHUNK)], rows[b],
                        ssem.at[b]).wait()

                load_idx(b, e0)
                descs.append(
                    pltpu.async_copy(g_hbm.at[sidx[b]], rows[b], gsem.at[b])
                )
            for b in range(NBUF):
                descs[b].wait()
                pltpu.async_copy(rows[b], acc_sp.at[didx[b]],
                                 ssem.at[b], add=True)
            return carry

        lax.fori_loop(0, NPAIRS, pair, 0)

        for b in range(NBUF):
            pltpu.make_async_copy(
                g_hbm.at[pl.ds(0, ECHUNK)], rows[b], ssem.at[b]
            ).wait()
        plsc.subcore_barrier()
        pltpu.sync_copy(acc_sp.at[pl.ds(r0, ROWS_PER_TILE)],
                        out_hbm.at[pl.ds(r0, ROWS_PER_TILE)])

    @pl.when(c == 0)
    def _():
        run(gl_hbm, outl_hbm)

    @pl.when(c == 1)
    def _():
        run(gr_hbm, outr_hbm)


_TCBLK = 512


def _tca_body(x_ref, w_ref, da_ref, db_ref, gl_ref, gr_ref, dis_ref):
    deg = da_ref[...] + db_ref[...] + 1.0
    dis = lax.rsqrt(deg)
    h = jnp.dot(x_ref[...], w_ref[...], preferred_element_type=jnp.float32)
    g = h * dis
    gl_ref[...] = g[:, :DH]
    gr_ref[...] = g[:, DH:]
    dis_ref[...] = dis


def _tc_transform(x_pad, W, degA, degB):
    grid = (NPAD // _TCBLK,)
    return pl.pallas_call(
        _tca_body,
        grid=grid,
        in_specs=[
            pl.BlockSpec((_TCBLK, D), lambda i: (i, 0)),
            pl.BlockSpec((D, D), lambda i: (0, 0)),
            pl.BlockSpec((_TCBLK, 1), lambda i: (i, 0)),
            pl.BlockSpec((_TCBLK, 1), lambda i: (i, 0)),
        ],
        out_specs=[
            pl.BlockSpec((_TCBLK, DH), lambda i: (i, 0)),
            pl.BlockSpec((_TCBLK, DH), lambda i: (i, 0)),
            pl.BlockSpec((_TCBLK, 1), lambda i: (i, 0)),
        ],
        out_shape=[
            jax.ShapeDtypeStruct((NPAD, DH), jnp.float32),
            jax.ShapeDtypeStruct((NPAD, DH), jnp.float32),
            jax.ShapeDtypeStruct((NPAD, 1), jnp.float32),
        ],
    )(x_pad, W, degA, degB)


def _tcb_body(al_ref, ar_ref, dis_ref, b_ref, o_ref):
    dis = dis_ref[...]
    b = b_ref[...]
    o_ref[:, :DH] = jnp.tanh(al_ref[...] * dis + b[:, :DH])
    o_ref[:, DH:] = jnp.tanh(ar_ref[...] * dis + b[:, DH:])


def _tc_final(accL, accR, dis, b2d):
    grid = (NPAD // _TCBLK,)
    return pl.pallas_call(
        _tcb_body,
        grid=grid,
        in_specs=[
            pl.BlockSpec((_TCBLK, DH), lambda i: (i, 0)),
            pl.BlockSpec((_TCBLK, DH), lambda i: (i, 0)),
            pl.BlockSpec((_TCBLK, 1), lambda i: (i, 0)),
            pl.BlockSpec((1, D), lambda i: (0, 0)),
        ],
        out_specs=pl.BlockSpec((_TCBLK, D), lambda i: (i, 0)),
        out_shape=jax.ShapeDtypeStruct((NPAD, D), jnp.float32),
    )(accL, accR, dis, b2d)


@jax.jit
def kernel(x, edge_index, W, b):
    src = edge_index[0].astype(jnp.int32)
    dst = edge_index[1].astype(jnp.int32)
    pad_idx = N + (jnp.arange(EPAD - E, dtype=jnp.int32) % (NPAD - N))
    src = jnp.concatenate([src, pad_idx])
    dst = jnp.concatenate([dst, pad_idx])
    x_pad = jnp.pad(x, ((0, NPAD - N), (0, 0)))
    zeros = jnp.zeros((NPAD,), jnp.float32)
    ones = jnp.ones((DEG_CHUNK,), jnp.float32)

    deg2 = _deg_kernel(dst, zeros, ones)
    degA = deg2[:NPAD].reshape(NPAD, 1)
    degB = deg2[NPAD:].reshape(NPAD, 1)

    gl, gr, dis = _tc_transform(x_pad, W, degA, degB)
    accL, accR = _scatter_kernel(gl, gr, src, dst)
    out = _tc_final(accL, accR, dis, b.reshape(1, D))
    return out[:N]


# speedup vs baseline: 2.0273x; 1.2230x over previous
import functools

import jax
import jax.numpy as jnp
from jax import lax
from jax.experimental import pallas as pl
from jax.experimental.pallas import tpu as pltpu
from jax.experimental.pallas import tpu_sc as plsc

N = 10000
E = 160000
D = 256
DH = 128
NPAD = 10240
EPAD = 163840
NC, NS = 2, 16
ROWS_PER_TILE = NPAD // NS
EDGES_PER_TILE = EPAD // NS
ECHUNK = 80
NCHUNKS = EDGES_PER_TILE // ECHUNK
NBUF = 4
NQUARTER = NCHUNKS // 4
EDGES_PER_QUARTER = NQUARTER * ECHUNK
NGROUPS = NQUARTER // NBUF
DEG_NBUF = 2
DEG_CHUNK = 128
DEG_EPT = EPAD // (NC * NS)
DEG_NCHUNKS = DEG_EPT // DEG_CHUNK
DEG_NPAIRS = DEG_NCHUNKS // DEG_NBUF

_MESH = plsc.VectorSubcoreMesh(core_axis_name="c", subcore_axis_name="s")


@functools.partial(
    pl.kernel,
    out_type=jax.ShapeDtypeStruct((NC * NPAD,), jnp.float32),
    mesh=_MESH,
    scratch_types=[
        pltpu.VMEM((DEG_EPT,), jnp.int32),
        [pltpu.VMEM((DEG_CHUNK,), jnp.int32) for _ in range(DEG_NBUF)],
        pltpu.VMEM((DEG_CHUNK,), jnp.float32),
        pltpu.VMEM_SHARED((NPAD,), jnp.float32),
        pltpu.SemaphoreType.DMA((DEG_NBUF,)),
    ],
)
def _deg_kernel(dst_hbm, zeros_hbm, ones_hbm, out_hbm, dstb, didx, ones_v,
                deg_sp, ssem):
    c = lax.axis_index("c")
    s = lax.axis_index("s")
    tid = c * NS + s
    r0 = s * ROWS_PER_TILE
    pltpu.sync_copy(zeros_hbm.at[pl.ds(r0, ROWS_PER_TILE)],
                    deg_sp.at[pl.ds(r0, ROWS_PER_TILE)])
    pltpu.sync_copy(ones_hbm, ones_v)
    e_base = pl.multiple_of(tid * DEG_EPT, 8)
    pltpu.sync_copy(dst_hbm.at[pl.ds(e_base, DEG_EPT)], dstb)
    plsc.subcore_barrier()

    def pair(k, carry):
        for b in range(DEG_NBUF):
            e0 = pl.multiple_of((k * DEG_NBUF + b) * DEG_CHUNK, 16)

            @pl.when(k > 0)
            def _(b=b):
                pltpu.make_async_copy(ones_hbm, ones_v, ssem.at[b]).wait()

            for j in range(DEG_CHUNK // 16):
                o = pl.multiple_of(e0 + j * 16, 16)
                didx[b][pl.ds(j * 16, 16)] = dstb[pl.ds(o, 16)]
            pltpu.async_copy(ones_v, deg_sp.at[didx[b]], ssem.at[b], add=True)
        return carry

    lax.fori_loop(0, DEG_NPAIRS, pair, 0)
    for b in range(DEG_NBUF):
        pltpu.make_async_copy(ones_hbm, ones_v, ssem.at[b]).wait()
    plsc.subcore_barrier()
    out0 = c * NPAD + r0
    pltpu.sync_copy(deg_sp.at[pl.ds(r0, ROWS_PER_TILE)],
                    out_hbm.at[pl.ds(out0, ROWS_PER_TILE)])


@functools.partial(
    pl.kernel,
    out_type=[
        jax.ShapeDtypeStruct((NPAD, DH), jnp.float32),
        jax.ShapeDtypeStruct((NPAD, DH), jnp.float32),
    ],
    mesh=_MESH,
    scratch_types=[
        pltpu.VMEM((EDGES_PER_QUARTER,), jnp.int32),
        pltpu.VMEM((EDGES_PER_QUARTER,), jnp.int32),
        [pltpu.VMEM((ECHUNK, DH), jnp.float32) for _ in range(NBUF)],
        [pltpu.VMEM((ECHUNK,), jnp.int32) for _ in range(NBUF)],
        [pltpu.VMEM((ECHUNK,), jnp.int32) for _ in range(NBUF)],
        pltpu.VMEM_SHARED((NPAD, DH), jnp.float32),
        pltpu.SemaphoreType.DMA((NBUF,)),
        pltpu.SemaphoreType.DMA((NBUF,)),
    ],
)
def _scatter_kernel(gl_hbm, gr_hbm, src_hbm, dst_hbm, outl_hbm, outr_hbm,
                    srcb, dstb, rows, sidx, didx, acc_sp, gsem, ssem):
    c = lax.axis_index("c")
    s = lax.axis_index("s")
    r0 = s * ROWS_PER_TILE

    def run(g_hbm, out_hbm):
        pltpu.sync_copy(g_hbm.at[pl.ds(r0, ROWS_PER_TILE)],
                        acc_sp.at[pl.ds(r0, ROWS_PER_TILE)])
        plsc.subcore_barrier()

        def load_idx(b, e0):
            for j in range(ECHUNK // 16):
                o = pl.multiple_of(e0 + j * 16, 16)
                sidx[b][pl.ds(j * 16, 16)] = srcb[pl.ds(o, 16)]
                didx[b][pl.ds(j * 16, 16)] = dstb[pl.ds(o, 16)]

        for q in range(4):
            e_base = pl.multiple_of(
                s * EDGES_PER_TILE + q * EDGES_PER_QUARTER, 8)
            pltpu.sync_copy(src_hbm.at[pl.ds(e_base, EDGES_PER_QUARTER)], srcb)
            pltpu.sync_copy(dst_hbm.at[pl.ds(e_base, EDGES_PER_QUARTER)], dstb)

            def group(k, carry, q=q):
                descs = []
                for b in range(NBUF):
                    e0 = pl.multiple_of((k * NBUF + b) * ECHUNK, 16)

                    if q == 0:
                        @pl.when(k > 0)
                        def _(b=b):
                            pltpu.make_async_copy(
                                g_hbm.at[pl.ds(0, ECHUNK)], rows[b],
                                ssem.at[b]).wait()
                    else:
                        pltpu.make_async_copy(
                            g_hbm.at[pl.ds(0, ECHUNK)], rows[b],
                            ssem.at[b]).wait()

                    load_idx(b, e0)
                    descs.append(
                        pltpu.async_copy(g_hbm.at[sidx[b]], rows[b],
                                         gsem.at[b])
                    )
                for b in range(NBUF):
                    descs[b].wait()
                    pltpu.async_copy(rows[b], acc_sp.at[didx[b]],
                                     ssem.at[b], add=True)
                return carry

            lax.fori_loop(0, NGROUPS, group, 0)

        for b in range(NBUF):
            pltpu.make_async_copy(
                g_hbm.at[pl.ds(0, ECHUNK)], rows[b], ssem.at[b]
            ).wait()
        plsc.subcore_barrier()
        pltpu.sync_copy(acc_sp.at[pl.ds(r0, ROWS_PER_TILE)],
                        out_hbm.at[pl.ds(r0, ROWS_PER_TILE)])

    @pl.when(c == 0)
    def _():
        run(gl_hbm, outl_hbm)

    @pl.when(c == 1)
    def _():
        run(gr_hbm, outr_hbm)


_TCBLK = 400


def _tca_body(x_ref, w_ref, da_ref, db_ref, gl_ref, gr_ref, dis_ref):
    deg = da_ref[...] + db_ref[...] + 1.0
    dis = lax.rsqrt(deg)
    h = jnp.dot(x_ref[...], w_ref[...], preferred_element_type=jnp.float32)
    g = h * dis
    gl_ref[...] = g[:, :DH]
    gr_ref[...] = g[:, DH:]
    dis_ref[...] = dis


def _tc_transform(x, W, degA, degB):
    grid = (N // _TCBLK,)
    return pl.pallas_call(
        _tca_body,
        grid=grid,
        in_specs=[
            pl.BlockSpec((_TCBLK, D), lambda i: (i, 0)),
            pl.BlockSpec((D, D), lambda i: (0, 0)),
            pl.BlockSpec((_TCBLK, 1), lambda i: (i, 0)),
            pl.BlockSpec((_TCBLK, 1), lambda i: (i, 0)),
        ],
        out_specs=[
            pl.BlockSpec((_TCBLK, DH), lambda i: (i, 0)),
            pl.BlockSpec((_TCBLK, DH), lambda i: (i, 0)),
            pl.BlockSpec((_TCBLK, 1), lambda i: (i, 0)),
        ],
        out_shape=[
            jax.ShapeDtypeStruct((NPAD, DH), jnp.float32),
            jax.ShapeDtypeStruct((NPAD, DH), jnp.float32),
            jax.ShapeDtypeStruct((NPAD, 1), jnp.float32),
        ],
    )(x, W, degA, degB)


def _tcb_body(al_ref, ar_ref, dis_ref, b_ref, o_ref):
    dis = dis_ref[...]
    b = b_ref[...]
    o_ref[:, :DH] = jnp.tanh(al_ref[...] * dis + b[:, :DH])
    o_ref[:, DH:] = jnp.tanh(ar_ref[...] * dis + b[:, DH:])


def _tc_final(accL, accR, dis, b2d):
    grid = (N // _TCBLK,)
    return pl.pallas_call(
        _tcb_body,
        grid=grid,
        in_specs=[
            pl.BlockSpec((_TCBLK, DH), lambda i: (i, 0)),
            pl.BlockSpec((_TCBLK, DH), lambda i: (i, 0)),
            pl.BlockSpec((_TCBLK, 1), lambda i: (i, 0)),
            pl.BlockSpec((1, D), lambda i: (0, 0)),
        ],
        out_specs=pl.BlockSpec((_TCBLK, D), lambda i: (i, 0)),
        out_shape=jax.ShapeDtypeStruct((N, D), jnp.float32),
    )(accL, accR, dis, b2d)


@jax.jit
def kernel(x, edge_index, W, b):
    src = edge_index[0].astype(jnp.int32)
    dst = edge_index[1].astype(jnp.int32)
    pad_idx = N + (jnp.arange(EPAD - E, dtype=jnp.int32) % (NPAD - N))
    src = jnp.concatenate([src, pad_idx])
    dst = jnp.concatenate([dst, pad_idx])
    zeros = jnp.zeros((NPAD,), jnp.float32)
    ones = jnp.ones((DEG_CHUNK,), jnp.float32)

    deg2 = _deg_kernel(dst, zeros, ones)
    degA = deg2[:NPAD].reshape(NPAD, 1)
    degB = deg2[NPAD:].reshape(NPAD, 1)

    gl, gr, dis = _tc_transform(x, W, degA, degB)
    accL, accR = _scatter_kernel(gl, gr, src, dst)
    return _tc_final(accL, accR, dis, b.reshape(1, D))


# speedup vs baseline: 2.0446x; 1.0086x over previous
import functools

import jax
import jax.numpy as jnp
from jax import lax
from jax.experimental import pallas as pl
from jax.experimental.pallas import tpu as pltpu
from jax.experimental.pallas import tpu_sc as plsc

N = 10000
E = 160000
D = 256
DH = 128
NPAD = 10240
EPAD = 163840
NC, NS = 2, 16
ROWS_PER_TILE = NPAD // NS
EDGES_PER_TILE = EPAD // NS
ECHUNK = 64
NCHUNKS = EDGES_PER_TILE // ECHUNK
NBUF = 5
NQUARTER = NCHUNKS // 4
EDGES_PER_QUARTER = NQUARTER * ECHUNK
NGROUPS = NQUARTER // NBUF
DEG_NBUF = 2
DEG_CHUNK = 128
DEG_EPT = EPAD // (NC * NS)
DEG_NCHUNKS = DEG_EPT // DEG_CHUNK
DEG_NPAIRS = DEG_NCHUNKS // DEG_NBUF

_MESH = plsc.VectorSubcoreMesh(core_axis_name="c", subcore_axis_name="s")


@functools.partial(
    pl.kernel,
    out_type=jax.ShapeDtypeStruct((NC * NPAD,), jnp.float32),
    mesh=_MESH,
    scratch_types=[
        pltpu.VMEM((DEG_EPT,), jnp.int32),
        [pltpu.VMEM((DEG_CHUNK,), jnp.int32) for _ in range(DEG_NBUF)],
        pltpu.VMEM((DEG_CHUNK,), jnp.float32),
        pltpu.VMEM_SHARED((NPAD,), jnp.float32),
        pltpu.SemaphoreType.DMA((DEG_NBUF,)),
    ],
)
def _deg_kernel(dst_hbm, zeros_hbm, ones_hbm, out_hbm, dstb, didx, ones_v,
                deg_sp, ssem):
    c = lax.axis_index("c")
    s = lax.axis_index("s")
    tid = c * NS + s
    r0 = s * ROWS_PER_TILE
    pltpu.sync_copy(zeros_hbm.at[pl.ds(r0, ROWS_PER_TILE)],
                    deg_sp.at[pl.ds(r0, ROWS_PER_TILE)])
    pltpu.sync_copy(ones_hbm, ones_v)
    e_base = pl.multiple_of(tid * DEG_EPT, 8)
    pltpu.sync_copy(dst_hbm.at[pl.ds(e_base, DEG_EPT)], dstb)
    plsc.subcore_barrier()

    def pair(k, carry):
        for b in range(DEG_NBUF):
            e0 = pl.multiple_of((k * DEG_NBUF + b) * DEG_CHUNK, 16)

            @pl.when(k > 0)
            def _(b=b):
                pltpu.make_async_copy(ones_hbm, ones_v, ssem.at[b]).wait()

            for j in range(DEG_CHUNK // 16):
                o = pl.multiple_of(e0 + j * 16, 16)
                didx[b][pl.ds(j * 16, 16)] = dstb[pl.ds(o, 16)]
            pltpu.async_copy(ones_v, deg_sp.at[didx[b]], ssem.at[b], add=True)
        return carry

    lax.fori_loop(0, DEG_NPAIRS, pair, 0)
    for b in range(DEG_NBUF):
        pltpu.make_async_copy(ones_hbm, ones_v, ssem.at[b]).wait()
    plsc.subcore_barrier()
    out0 = c * NPAD + r0
    pltpu.sync_copy(deg_sp.at[pl.ds(r0, ROWS_PER_TILE)],
                    out_hbm.at[pl.ds(out0, ROWS_PER_TILE)])


@functools.partial(
    pl.kernel,
    out_type=[
        jax.ShapeDtypeStruct((NPAD, DH), jnp.float32),
        jax.ShapeDtypeStruct((NPAD, DH), jnp.float32),
    ],
    mesh=_MESH,
    scratch_types=[
        pltpu.VMEM((EDGES_PER_QUARTER,), jnp.int32),
        pltpu.VMEM((EDGES_PER_QUARTER,), jnp.int32),
        [pltpu.VMEM((ECHUNK, DH), jnp.float32) for _ in range(NBUF)],
        [pltpu.VMEM((ECHUNK,), jnp.int32) for _ in range(NBUF)],
        [pltpu.VMEM((ECHUNK,), jnp.int32) for _ in range(NBUF)],
        pltpu.VMEM_SHARED((NPAD, DH), jnp.float32),
        pltpu.SemaphoreType.DMA((NBUF,)),
        pltpu.SemaphoreType.DMA((NBUF,)),
    ],
)
def _scatter_kernel(gl_hbm, gr_hbm, src_hbm, dst_hbm, outl_hbm, outr_hbm,
                    srcb, dstb, rows, sidx, didx, acc_sp, gsem, ssem):
    c = lax.axis_index("c")
    s = lax.axis_index("s")
    r0 = s * ROWS_PER_TILE

    def run(g_hbm, out_hbm):
        pltpu.sync_copy(g_hbm.at[pl.ds(r0, ROWS_PER_TILE)],
                        acc_sp.at[pl.ds(r0, ROWS_PER_TILE)])
        plsc.subcore_barrier()

        def load_idx(b, e0):
            for j in range(ECHUNK // 16):
                o = pl.multiple_of(e0 + j * 16, 16)
                sidx[b][pl.ds(j * 16, 16)] = srcb[pl.ds(o, 16)]
                didx[b][pl.ds(j * 16, 16)] = dstb[pl.ds(o, 16)]

        for q in range(4):
            e_base = pl.multiple_of(
                s * EDGES_PER_TILE + q * EDGES_PER_QUARTER, 8)
            pltpu.sync_copy(src_hbm.at[pl.ds(e_base, EDGES_PER_QUARTER)], srcb)
            pltpu.sync_copy(dst_hbm.at[pl.ds(e_base, EDGES_PER_QUARTER)], dstb)

            def group(k, carry, q=q):
                descs = []
                for b in range(NBUF):
                    e0 = pl.multiple_of((k * NBUF + b) * ECHUNK, 16)

                    if q == 0:
                        @pl.when(k > 0)
                        def _(b=b):
                            pltpu.make_async_copy(
                                g_hbm.at[pl.ds(0, ECHUNK)], rows[b],
                                ssem.at[b]).wait()
                    else:
                        pltpu.make_async_copy(
                            g_hbm.at[pl.ds(0, ECHUNK)], rows[b],
                            ssem.at[b]).wait()

                    load_idx(b, e0)
                    descs.append(
                        pltpu.async_copy(g_hbm.at[sidx[b]], rows[b],
                                         gsem.at[b])
                    )
                for b in range(NBUF):
                    descs[b].wait()
                    pltpu.async_copy(rows[b], acc_sp.at[didx[b]],
                                     ssem.at[b], add=True)
                return carry

            lax.fori_loop(0, NGROUPS, group, 0)

        for b in range(NBUF):
            pltpu.make_async_copy(
                g_hbm.at[pl.ds(0, ECHUNK)], rows[b], ssem.at[b]
            ).wait()
        plsc.subcore_barrier()
        pltpu.sync_copy(acc_sp.at[pl.ds(r0, ROWS_PER_TILE)],
                        out_hbm.at[pl.ds(r0, ROWS_PER_TILE)])

    @pl.when(c == 0)
    def _():
        run(gl_hbm, outl_hbm)

    @pl.when(c == 1)
    def _():
        run(gr_hbm, outr_hbm)


_TCBLK = 400


def _tca_body(x_ref, w_ref, da_ref, db_ref, gl_ref, gr_ref, dis_ref):
    deg = da_ref[...] + db_ref[...] + 1.0
    dis = lax.rsqrt(deg)
    h = jnp.dot(x_ref[...], w_ref[...], preferred_element_type=jnp.float32)
    g = h * dis
    gl_ref[...] = g[:, :DH]
    gr_ref[...] = g[:, DH:]
    dis_ref[...] = dis


def _tc_transform(x, W, degA, degB):
    grid = (N // _TCBLK,)
    return pl.pallas_call(
        _tca_body,
        grid=grid,
        in_specs=[
            pl.BlockSpec((_TCBLK, D), lambda i: (i, 0)),
            pl.BlockSpec((D, D), lambda i: (0, 0)),
            pl.BlockSpec((_TCBLK, 1), lambda i: (i, 0)),
            pl.BlockSpec((_TCBLK, 1), lambda i: (i, 0)),
        ],
        out_specs=[
            pl.BlockSpec((_TCBLK, DH), lambda i: (i, 0)),
            pl.BlockSpec((_TCBLK, DH), lambda i: (i, 0)),
            pl.BlockSpec((_TCBLK, 1), lambda i: (i, 0)),
        ],
        out_shape=[
            jax.ShapeDtypeStruct((NPAD, DH), jnp.float32),
            jax.ShapeDtypeStruct((NPAD, DH), jnp.float32),
            jax.ShapeDtypeStruct((NPAD, 1), jnp.float32),
        ],
    )(x, W, degA, degB)


def _tcb_body(al_ref, ar_ref, dis_ref, b_ref, o_ref):
    dis = dis_ref[...]
    b = b_ref[...]
    o_ref[:, :DH] = jnp.tanh(al_ref[...] * dis + b[:, :DH])
    o_ref[:, DH:] = jnp.tanh(ar_ref[...] * dis + b[:, DH:])


def _tc_final(accL, accR, dis, b2d):
    grid = (N // _TCBLK,)
    return pl.pallas_call(
        _tcb_body,
        grid=grid,
        in_specs=[
            pl.BlockSpec((_TCBLK, DH), lambda i: (i, 0)),
            pl.BlockSpec((_TCBLK, DH), lambda i: (i, 0)),
            pl.BlockSpec((_TCBLK, 1), lambda i: (i, 0)),
            pl.BlockSpec((1, D), lambda i: (0, 0)),
        ],
        out_specs=pl.BlockSpec((_TCBLK, D), lambda i: (i, 0)),
        out_shape=jax.ShapeDtypeStruct((N, D), jnp.float32),
    )(accL, accR, dis, b2d)


@jax.jit
def kernel(x, edge_index, W, b):
    src = edge_index[0].astype(jnp.int32)
    dst = edge_index[1].astype(jnp.int32)
    pad_idx = N + (jnp.arange(EPAD - E, dtype=jnp.int32) % (NPAD - N))
    src = jnp.concatenate([src, pad_idx])
    dst = jnp.concatenate([dst, pad_idx])
    zeros = jnp.zeros((NPAD,), jnp.float32)
    ones = jnp.ones((DEG_CHUNK,), jnp.float32)

    deg2 = _deg_kernel(dst, zeros, ones)
    degA = deg2[:NPAD].reshape(NPAD, 1)
    degB = deg2[NPAD:].reshape(NPAD, 1)

    gl, gr, dis = _tc_transform(x, W, degA, degB)
    accL, accR = _scatter_kernel(gl, gr, src, dst)
    return _tc_final(accL, accR, dis, b.reshape(1, D))
